# Initial kernel scaffold; baseline (speedup 1.0000x reference)
import jax
import jax.numpy as jnp
from jax.experimental import pallas as pl


def kernel(positions, features, W_e0, W_e1, W_inf, W_x0, W_x1, W_tp):
    raise NotImplementedError("write your pallas kernel here")


# speedup vs baseline: 14.2367x; 14.2367x over previous
import math

import jax
import jax.numpy as jnp
import numpy as np
from jax.experimental import pallas as pl
from jax.experimental.pallas import tpu as pltpu

N = 512
N_VEC = 4
N_FEAT = 32
D = 64


def _egcl_kernel(pdup_ref, featdup_ref, A_ref, B_ref, C_ref, We1_ref,
                 Winf_ref, Wx0_ref, Wx1_ref, Wt_ref, M_ref,
                 mi_ref, vec_ref, fb_ref, r_ref):
    i = pl.program_id(0)

    @pl.when(i == 0)
    def _precompute():
        fb_ref[...] = jnp.dot(featdup_ref[...], B_ref[...],
                              preferred_element_type=jnp.float32)
        r_ref[...] = jnp.dot(featdup_ref[0:N, :], C_ref[...],
                             preferred_element_type=jnp.float32)

    ps = pdup_ref[pl.ds(i + 1, N), :]
    pr = pdup_ref[pl.ds(i, 1), :]
    v = ps - pr
    vx = v[:, 0:4]
    vy = v[:, 4:8]
    vz = v[:, 8:12]
    len2 = vx * vx + vy * vy + vz * vz
    inv = jnp.where(len2 > 0, jax.lax.rsqrt(len2), 0.0)
    lengths = len2 * inv

    fB = fb_ref[pl.ds(i + 1, N), :]
    bias = r_ref[pl.ds(i, 1), :]
    h0 = jnp.dot(lengths, A_ref[...], preferred_element_type=jnp.float32)
    h0 = jax.nn.silu(h0 + fB + bias)
    m = jax.nn.silu(jnp.dot(h0, We1_ref[...],
                            preferred_element_type=jnp.float32))

    e = jax.nn.sigmoid(jnp.dot(m, Winf_ref[...],
                               preferred_element_type=jnp.float32))
    row = jax.lax.broadcasted_iota(jnp.int32, (N, 1), 0)
    e = jnp.where(row < N - 1, e, 0.0)
    mi_ref[0] = jnp.sum(m * e, axis=0, keepdims=True)

    hx = jax.nn.silu(jnp.dot(m, Wx0_ref[...],
                             preferred_element_type=jnp.float32))
    phi = jax.nn.silu(jnp.dot(hx, Wx1_ref[...],
                              preferred_element_type=jnp.float32))
    T = jnp.dot(phi, Wt_ref[...], preferred_element_type=jnp.float32)

    ux = vx * inv
    uy = vy * inv
    uz = vz * inv
    P = jnp.concatenate(
        [T * jnp.concatenate([ux, ux, ux, ux], axis=1),
         T * jnp.concatenate([uy, uy, uy, uy], axis=1),
         T * jnp.concatenate([uz, uz, uz, uz], axis=1)], axis=1)
    out12 = jnp.dot(P, M_ref[...], preferred_element_type=jnp.float32)
    vec_ref[0] = out12[0:N - 1, :]


def kernel(positions, features, W_e0, W_e1, W_inf, W_x0, W_x1, W_tp):
    f32 = jnp.float32

    p12 = positions.transpose(0, 2, 1).reshape(N, 3 * N_VEC)
    pdup = jnp.concatenate([p12, p12], axis=0)
    featdup = jnp.concatenate([features, features], axis=0)

    s0 = 1.0 / math.sqrt(N_VEC + 2 * N_FEAT)
    A = W_e0[0:N_VEC] * s0
    B = W_e0[N_VEC:N_VEC + N_FEAT] * s0
    C = W_e0[N_VEC + N_FEAT:] * s0
    sD = 1.0 / math.sqrt(D)
    We1 = W_e1 * sD
    Winf = W_inf * sD
    Wx0 = W_x0 * sD
    Wx1 = W_x1 * sD
    Wt = (W_tp.transpose(0, 2, 1).reshape(D, N_VEC * N_VEC)
          * (math.sqrt(3.0) / math.sqrt(D * N_VEC)))
    Mnp = np.zeros((3 * N_VEC * N_VEC, 3 * N_VEC), dtype=np.float32)
    for c in range(3):
        for u in range(N_VEC):
            for k in range(N_VEC):
                Mnp[c * 16 + 4 * u + k, 3 * u + c] = 1.0
    M = jnp.asarray(Mnp)

    full = lambda shape: pl.BlockSpec(shape, lambda i: (0,) * len(shape))
    mi3, vec = pl.pallas_call(
        _egcl_kernel,
        grid=(N,),
        in_specs=[
            full((2 * N, 3 * N_VEC)),
            full((2 * N, N_FEAT)),
            full((N_VEC, D)),
            full((N_FEAT, D)),
            full((N_FEAT, D)),
            full((D, D)),
            full((D, 1)),
            full((D, D)),
            full((D, D)),
            full((D, N_VEC * N_VEC)),
            full((3 * N_VEC * N_VEC, 3 * N_VEC)),
        ],
        out_specs=[
            pl.BlockSpec((1, 1, D), lambda i: (i, 0, 0)),
            pl.BlockSpec((1, N - 1, 3 * N_VEC), lambda i: (i, 0, 0)),
        ],
        out_shape=[
            jax.ShapeDtypeStruct((N, 1, D), f32),
            jax.ShapeDtypeStruct((N, N - 1, 3 * N_VEC), f32),
        ],
        scratch_shapes=[
            pltpu.VMEM((2 * N, D), f32),
            pltpu.VMEM((N, D), f32),
        ],
    )(pdup, featdup, A, B, C, We1, Winf, Wx0, Wx1, Wt, M)

    m_i = mi3.reshape(N, D)
    vec_out = vec.reshape(N, N - 1, N_VEC, 3)
    return m_i, vec_out
